# initial kernel scaffold (unmeasured)
import jax
import jax.numpy as jnp
from jax import lax
from jax.experimental import pallas as pl
from jax.experimental.pallas import tpu as pltpu

X_SIZE = 2


def kernel(x, assign, W1, W2):
    t_per, d = x.shape
    e_loc, _, f = W1.shape

    assign2d = assign.reshape(t_per, 1)

    def body(x_ref, a_ref, w1_ref, w2_ref, out_ref,
             xsend, xrecv, arecv, res_send, res_recv, sems):
        my_x = lax.axis_index("x")
        my_y = lax.axis_index("y")
        my_z = lax.axis_index("z")
        peer = (1 - my_x, my_y, my_z)

        barrier = pltpu.get_barrier_semaphore()
        pl.semaphore_signal(barrier, inc=1, device_id=peer,
                            device_id_type=pl.DeviceIdType.MESH)
        pl.semaphore_wait(barrier, 1)

        xsend[...] = x_ref[...].astype(jnp.bfloat16)
        rdma_x = pltpu.make_async_remote_copy(
            src_ref=xsend, dst_ref=xrecv,
            send_sem=sems.at[0], recv_sem=sems.at[1],
            device_id=peer, device_id_type=pl.DeviceIdType.MESH)
        rdma_x.start()
        rdma_a = pltpu.make_async_remote_copy(
            src_ref=a_ref, dst_ref=arecv,
            send_sem=sems.at[2], recv_sem=sems.at[3],
            device_id=peer, device_id_type=pl.DeviceIdType.MESH)
        rdma_a.start()

        def contrib(xb, a2d):
            acc = jnp.zeros((t_per, d), jnp.float32)
            for e in range(e_loc):
                g = my_x * e_loc + e
                xe = jnp.where(a2d == g, xb, jnp.bfloat16(0))
                h = lax.dot_general(
                    xe, w1_ref[e].astype(jnp.bfloat16),
                    (((1,), (0,)), ((), ())),
                    preferred_element_type=jnp.float32)
                h = jnp.maximum(h, 0.0).astype(jnp.bfloat16)
                acc += lax.dot_general(
                    h, w2_ref[e].astype(jnp.bfloat16),
                    (((1,), (0,)), ((), ())),
                    preferred_element_type=jnp.float32)
            return acc

        out_ref[...] = contrib(xsend[...], a_ref[...])

        rdma_x.wait()
        rdma_a.wait()
        res_send[...] = contrib(xrecv[...], arecv[...]).astype(jnp.bfloat16)
        rdma_r = pltpu.make_async_remote_copy(
            src_ref=res_send, dst_ref=res_recv,
            send_sem=sems.at[4], recv_sem=sems.at[5],
            device_id=peer, device_id_type=pl.DeviceIdType.MESH)
        rdma_r.start()
        rdma_r.wait()

        out_ref[...] += res_recv[...].astype(jnp.float32)

    return pl.pallas_call(
        body,
        out_shape=jax.ShapeDtypeStruct((t_per, d), jnp.float32),
        in_specs=[
            pl.BlockSpec(memory_space=pltpu.VMEM),
            pl.BlockSpec(memory_space=pltpu.VMEM),
            pl.BlockSpec(memory_space=pltpu.VMEM),
            pl.BlockSpec(memory_space=pltpu.VMEM),
        ],
        out_specs=pl.BlockSpec(memory_space=pltpu.VMEM),
        scratch_shapes=[
            pltpu.VMEM((t_per, d), jnp.bfloat16),
            pltpu.VMEM((t_per, d), jnp.bfloat16),
            pltpu.VMEM((t_per, 1), jnp.int32),
            pltpu.VMEM((t_per, d), jnp.bfloat16),
            pltpu.VMEM((t_per, d), jnp.bfloat16),
            pltpu.SemaphoreType.DMA((6,)),
        ],
        compiler_params=pltpu.CompilerParams(collective_id=0),
    )(x, assign2d, W1, W2)


# baseline (device time: 93949 ns/iter reference)
import jax
import jax.numpy as jnp
from jax import lax
from jax.experimental import pallas as pl
from jax.experimental.pallas import tpu as pltpu

X_SIZE = 2


def kernel(x, assign, W1, W2):
    t_per, d = x.shape
    e_loc, _, f = W1.shape

    assign2d = assign.reshape(t_per, 1)

    def body(x_ref, a_ref, w1_ref, w2_ref, out_ref,
             xsend, xrecv, arecv, res_send, res_recv, sems):
        my_x = lax.axis_index("x")
        my_y = lax.axis_index("y")
        my_z = lax.axis_index("z")
        peer = (1 - my_x, my_y, my_z)

        barrier = pltpu.get_barrier_semaphore()
        pl.semaphore_signal(barrier, inc=1, device_id=peer,
                            device_id_type=pl.DeviceIdType.MESH)
        pl.semaphore_wait(barrier, 1)

        xsend[...] = x_ref[...].astype(jnp.bfloat16)
        rdma_x = pltpu.make_async_remote_copy(
            src_ref=xsend, dst_ref=xrecv,
            send_sem=sems.at[0], recv_sem=sems.at[1],
            device_id=peer, device_id_type=pl.DeviceIdType.MESH)
        rdma_x.start()
        rdma_a = pltpu.make_async_remote_copy(
            src_ref=a_ref, dst_ref=arecv,
            send_sem=sems.at[2], recv_sem=sems.at[3],
            device_id=peer, device_id_type=pl.DeviceIdType.MESH)
        rdma_a.start()

        def contrib(xb, a2d):
            acc = jnp.zeros((t_per, d), jnp.float32)
            for e in range(e_loc):
                g = my_x * e_loc + e
                xe = jnp.where(a2d == g, xb, jnp.bfloat16(0))
                h = lax.dot_general(
                    xe, w1_ref[e].astype(jnp.bfloat16),
                    (((1,), (0,)), ((), ())),
                    preferred_element_type=jnp.float32)
                h = jnp.maximum(h, 0.0).astype(jnp.bfloat16)
                acc += lax.dot_general(
                    h, w2_ref[e].astype(jnp.bfloat16),
                    (((1,), (0,)), ((), ())),
                    preferred_element_type=jnp.float32)
            return acc

        out_ref[...] = contrib(xsend[...], a_ref[...])

        rdma_x.wait()
        rdma_a.wait()
        res_send[...] = contrib(xrecv[...], arecv[...]).astype(jnp.bfloat16)
        rdma_r = pltpu.make_async_remote_copy(
            src_ref=res_send, dst_ref=res_recv,
            send_sem=sems.at[4], recv_sem=sems.at[5],
            device_id=peer, device_id_type=pl.DeviceIdType.MESH)
        rdma_r.start()
        rdma_r.wait()

        out_ref[...] += res_recv[...].astype(jnp.float32)

    return pl.pallas_call(
        body,
        out_shape=jax.ShapeDtypeStruct((t_per, d), jnp.float32),
        in_specs=[
            pl.BlockSpec(memory_space=pltpu.VMEM),
            pl.BlockSpec(memory_space=pltpu.VMEM),
            pl.BlockSpec(memory_space=pltpu.VMEM),
            pl.BlockSpec(memory_space=pltpu.VMEM),
        ],
        out_specs=pl.BlockSpec(memory_space=pltpu.VMEM),
        scratch_shapes=[
            pltpu.VMEM((t_per, d), jnp.bfloat16),
            pltpu.VMEM((t_per, d), jnp.bfloat16),
            pltpu.VMEM((t_per, 1), jnp.int32),
            pltpu.VMEM((t_per, d), jnp.bfloat16),
            pltpu.VMEM((t_per, d), jnp.bfloat16),
            pltpu.SemaphoreType.DMA((6,)),
        ],
        compiler_params=pltpu.CompilerParams(
            collective_id=0,
            vmem_limit_bytes=100 * 1024 * 1024,
        ),
    )(x, assign2d, W1, W2)


# device time: 80358 ns/iter; 1.1691x vs baseline; 1.1691x over previous
import jax
import jax.numpy as jnp
from jax import lax
from jax.experimental import pallas as pl
from jax.experimental.pallas import tpu as pltpu

X_SIZE = 2
NC = 4


def kernel(x, assign, W1, W2):
    t_per, d = x.shape
    e_loc, _, f = W1.shape
    tc = t_per // NC

    assign2d = assign.reshape(t_per, 1)

    def body(x_ref, a_ref, w1_ref, w2_ref, out_ref,
             xsend, xrecv, arecv, res_send, res_recv,
             sems, rs_sems, rr_sems):
        my_x = lax.axis_index("x")
        my_y = lax.axis_index("y")
        my_z = lax.axis_index("z")
        peer = (1 - my_x, my_y, my_z)

        barrier = pltpu.get_barrier_semaphore()
        pl.semaphore_signal(barrier, inc=1, device_id=peer,
                            device_id_type=pl.DeviceIdType.MESH)
        pl.semaphore_wait(barrier, 1)

        xsend[...] = x_ref[...].astype(jnp.bfloat16)
        rdma_x = pltpu.make_async_remote_copy(
            src_ref=xsend, dst_ref=xrecv,
            send_sem=sems.at[0], recv_sem=sems.at[1],
            device_id=peer, device_id_type=pl.DeviceIdType.MESH)
        rdma_x.start()
        rdma_a = pltpu.make_async_remote_copy(
            src_ref=a_ref, dst_ref=arecv,
            send_sem=sems.at[2], recv_sem=sems.at[3],
            device_id=peer, device_id_type=pl.DeviceIdType.MESH)
        rdma_a.start()

        res_rdmas = [
            pltpu.make_async_remote_copy(
                src_ref=res_send.at[c], dst_ref=res_recv.at[c],
                send_sem=rs_sems.at[c], recv_sem=rr_sems.at[c],
                device_id=peer, device_id_type=pl.DeviceIdType.MESH)
            for c in range(NC)
        ]

        def contrib(xb, a2d):
            acc = jnp.zeros(xb.shape, jnp.float32)
            for e in range(e_loc):
                g = my_x * e_loc + e
                xe = jnp.where(a2d == g, xb, jnp.bfloat16(0))
                h = lax.dot_general(
                    xe, w1_ref[e].astype(jnp.bfloat16),
                    (((1,), (0,)), ((), ())),
                    preferred_element_type=jnp.float32)
                h = jnp.maximum(h, 0.0).astype(jnp.bfloat16)
                acc += lax.dot_general(
                    h, w2_ref[e].astype(jnp.bfloat16),
                    (((1,), (0,)), ((), ())),
                    preferred_element_type=jnp.float32)
            return acc

        for c in range(NC - 1):
            sl = pl.ds(c * tc, tc)
            out_ref[sl, :] = contrib(xsend[sl, :], a_ref[sl, :])

        rdma_x.wait()
        rdma_a.wait()
        for c in range(NC):
            sl = pl.ds(c * tc, tc)
            res_send[c] = contrib(xrecv[sl, :], arecv[sl, :]).astype(
                jnp.bfloat16)
            res_rdmas[c].start()

        sl = pl.ds((NC - 1) * tc, tc)
        out_ref[sl, :] = contrib(xsend[sl, :], a_ref[sl, :])

        for c in range(NC):
            res_rdmas[c].wait_recv()
            sl = pl.ds(c * tc, tc)
            out_ref[sl, :] += res_recv[c].astype(jnp.float32)
        for c in range(NC):
            res_rdmas[c].wait_send()

    return pl.pallas_call(
        body,
        out_shape=jax.ShapeDtypeStruct((t_per, d), jnp.float32),
        in_specs=[
            pl.BlockSpec(memory_space=pltpu.VMEM),
            pl.BlockSpec(memory_space=pltpu.VMEM),
            pl.BlockSpec(memory_space=pltpu.VMEM),
            pl.BlockSpec(memory_space=pltpu.VMEM),
        ],
        out_specs=pl.BlockSpec(memory_space=pltpu.VMEM),
        scratch_shapes=[
            pltpu.VMEM((t_per, d), jnp.bfloat16),
            pltpu.VMEM((t_per, d), jnp.bfloat16),
            pltpu.VMEM((t_per, 1), jnp.int32),
            pltpu.VMEM((NC, tc, d), jnp.bfloat16),
            pltpu.VMEM((NC, tc, d), jnp.bfloat16),
            pltpu.SemaphoreType.DMA((4,)),
            pltpu.SemaphoreType.DMA((NC,)),
            pltpu.SemaphoreType.DMA((NC,)),
        ],
        compiler_params=pltpu.CompilerParams(
            collective_id=0,
            vmem_limit_bytes=100 * 1024 * 1024,
        ),
    )(x, assign2d, W1, W2)
